# baseline (speedup 1.0000x reference)
import functools

import jax
import jax.numpy as jnp
from jax import lax
from jax.experimental import pallas as pl
from jax.experimental.pallas import tpu as pltpu
from jax.experimental.pallas import tpu_sc as plsc

_HID = 128
_VOCAB = 32
_NCORES = 2
_NSUB = 16
_NW = _NCORES * _NSUB
_C = 128
_W = 8
_NBUF = 4
_LEAD = 3


def _sc_embed(ids2d, table):
    n_rows = ids2d.shape[0] * _C
    b_per_w = n_rows // _NW
    chunks = b_per_w // _C
    wins = chunks // _W
    mesh = plsc.VectorSubcoreMesh(core_axis_name="c", subcore_axis_name="s")

    @functools.partial(
        pl.kernel,
        mesh=mesh,
        out_type=jax.ShapeDtypeStruct((n_rows, _HID), jnp.float32),
        scratch_types=[
            pltpu.VMEM((_W, _C), jnp.int32),
            pltpu.VMEM((_NBUF, _C, _HID), jnp.float32),
            pltpu.VMEM_SHARED((_VOCAB, _HID), jnp.float32),
        ] + [pltpu.SemaphoreType.DMA] * (2 * _NBUF),
    )
    def run(ids_hbm, table_hbm, out_hbm, idx_v, rows_v, table_s, *sems):
        sg = sems[:_NBUF]
        so = sems[_NBUF:]
        wid = lax.axis_index("s") * _NCORES + lax.axis_index("c")
        row0 = wid * b_per_w
        irow0 = row0 // _C

        @pl.when(lax.axis_index("s") == 0)
        def _():
            pltpu.sync_copy(table_hbm, table_s)

        plsc.subcore_barrier()

        def fire_gather(b, j):
            pltpu.async_copy(table_s.at[idx_v.at[j]], rows_v.at[b], sg[b])

        def wait_gather(b, j):
            pltpu.make_async_copy(
                table_s.at[idx_v.at[j]], rows_v.at[b], sg[b]).wait()

        def fire_out(b, chunk):
            pltpu.async_copy(
                rows_v.at[b], out_hbm.at[pl.ds((irow0 + chunk) * _C, _C)],
                so[b])

        def wait_out(b, chunk):
            pltpu.make_async_copy(
                rows_v.at[b], out_hbm.at[pl.ds((irow0 + chunk) * _C, _C)],
                so[b]).wait()

        def body(q, carry):
            c0 = q * _W
            pltpu.sync_copy(
                ids_hbm.at[pl.ds(pl.multiple_of(irow0 + c0, _W), _W)],
                idx_v)
            for j in range(_W):
                b = j % _NBUF
                if j >= _NBUF:
                    wait_out(b, c0 + j - _NBUF)
                else:
                    @pl.when(q > 0)
                    def _():
                        wait_out(b, c0 + j - _NBUF)
                fire_gather(b, j)
                if j >= _LEAD:
                    jd = j - _LEAD
                    wait_gather(jd % _NBUF, jd)
                    fire_out(jd % _NBUF, c0 + jd)
            for j in range(_W - _LEAD, _W):
                wait_gather(j % _NBUF, j)
                fire_out(j % _NBUF, c0 + j)
            return carry

        lax.fori_loop(0, wins, body, 0)
        for k in range(_NBUF):
            c = chunks - _NBUF + k
            wait_out(c % _NBUF, c)

    return run(ids2d, table)


def kernel(input_ids, attention_mask, emb_table):
    del attention_mask
    b, l = input_ids.shape
    n = b * l
    ids2d = input_ids.astype(jnp.int32).reshape(n // _C, _C)
    out = _sc_embed(ids2d, emb_table)
    return out.reshape(b, l, _HID)


# speedup vs baseline: 1.0521x; 1.0521x over previous
import functools

import jax
import jax.numpy as jnp
from jax import lax
from jax.experimental import pallas as pl
from jax.experimental.pallas import tpu as pltpu
from jax.experimental.pallas import tpu_sc as plsc

_HID = 128
_VOCAB = 32
_NCORES = 2
_NSUB = 16
_NW = _NCORES * _NSUB
_C = 128
_NBUF = 5
_LEAD = 2


def _sc_embed(ids2d, table):
    n_rows = ids2d.shape[0] * _C
    b_per_w = n_rows // _NW
    chunks = b_per_w // _C
    quads = chunks // _NBUF
    mesh = plsc.VectorSubcoreMesh(core_axis_name="c", subcore_axis_name="s")

    @functools.partial(
        pl.kernel,
        mesh=mesh,
        out_type=jax.ShapeDtypeStruct((n_rows, _HID), jnp.float32),
        scratch_types=[
            pltpu.VMEM((_NBUF, _C), jnp.int32),
            pltpu.VMEM((_NBUF, _C, _HID), jnp.float32),
            pltpu.VMEM_SHARED((_VOCAB, _HID), jnp.float32),
        ] + [pltpu.SemaphoreType.DMA] * (2 * _NBUF),
    )
    def run(ids_hbm, table_hbm, out_hbm, idx_v, rows_v, table_s, *sems):
        sg = sems[:_NBUF]
        so = sems[_NBUF:]
        wid = lax.axis_index("s") * _NCORES + lax.axis_index("c")
        row0 = wid * b_per_w
        irow0 = row0 // _C

        @pl.when(lax.axis_index("s") == 0)
        def _():
            pltpu.sync_copy(table_hbm, table_s)

        plsc.subcore_barrier()

        def load_ids(b, chunk):
            pltpu.sync_copy(ids_hbm.at[irow0 + chunk], idx_v.at[b])

        def fire_gather(b):
            pltpu.async_copy(table_s.at[idx_v.at[b]], rows_v.at[b], sg[b])

        def wait_gather(b):
            pltpu.make_async_copy(
                table_s.at[idx_v.at[b]], rows_v.at[b], sg[b]).wait()

        def fire_out(b, chunk):
            pltpu.async_copy(
                rows_v.at[b], out_hbm.at[pl.ds((irow0 + chunk) * _C, _C)],
                so[b])

        def wait_out(b, chunk):
            pltpu.make_async_copy(
                rows_v.at[b], out_hbm.at[pl.ds((irow0 + chunk) * _C, _C)],
                so[b]).wait()

        for c in range(_LEAD):
            load_ids(c, c)
            fire_gather(c)

        lag = _NBUF - _LEAD

        def body(q, carry):
            c0 = q * _NBUF
            for b in range(_NBUF):
                c = c0 + b
                wait_gather(b)
                fire_out(b, c)
                bn = (b + _LEAD) % _NBUF
                @pl.when(c >= lag)
                def _():
                    wait_out(bn, c - lag)

                @pl.when(c + _LEAD < chunks)
                def _():
                    load_ids(bn, c + _LEAD)
                    fire_gather(bn)
            return carry

        lax.fori_loop(0, quads, body, 0)
        for k in range(lag):
            c = chunks - lag + k
            wait_out(c % _NBUF, c)

    return run(ids2d, table)


def kernel(input_ids, attention_mask, emb_table):
    del attention_mask
    b, l = input_ids.shape
    n = b * l
    ids2d = input_ids.astype(jnp.int32).reshape(n // _C, _C)
    out = _sc_embed(ids2d, emb_table)
    return out.reshape(b, l, _HID)


# speedup vs baseline: 1.0844x; 1.0307x over previous
import functools

import jax
import jax.numpy as jnp
from jax import lax
from jax.experimental import pallas as pl
from jax.experimental.pallas import tpu as pltpu
from jax.experimental.pallas import tpu_sc as plsc

_HID = 128
_VOCAB = 32
_NCORES = 2
_NSUB = 16
_NW = _NCORES * _NSUB
_C = 128
_NBUF = 5
_LEAD = 3


def _sc_embed(ids2d, table):
    n_rows = ids2d.shape[0] * _C
    b_per_w = n_rows // _NW
    chunks = b_per_w // _C
    quads = chunks // _NBUF
    mesh = plsc.VectorSubcoreMesh(core_axis_name="c", subcore_axis_name="s")

    @functools.partial(
        pl.kernel,
        mesh=mesh,
        out_type=jax.ShapeDtypeStruct((n_rows, _HID), jnp.float32),
        scratch_types=[
            pltpu.VMEM((_NBUF, _C), jnp.int32),
            pltpu.VMEM((_NBUF, _C, _HID), jnp.float32),
            pltpu.VMEM_SHARED((_VOCAB, _HID), jnp.float32),
        ] + [pltpu.SemaphoreType.DMA] * (2 * _NBUF),
    )
    def run(ids_hbm, table_hbm, out_hbm, idx_v, rows_v, table_s, *sems):
        sg = sems[:_NBUF]
        so = sems[_NBUF:]
        wid = lax.axis_index("s") * _NCORES + lax.axis_index("c")
        row0 = wid * b_per_w
        irow0 = row0 // _C

        @pl.when(lax.axis_index("s") == 0)
        def _():
            pltpu.sync_copy(table_hbm, table_s)

        plsc.subcore_barrier()

        def load_ids(b, chunk):
            pltpu.sync_copy(ids_hbm.at[irow0 + chunk], idx_v.at[b])

        def fire_gather(b):
            pltpu.async_copy(table_s.at[idx_v.at[b]], rows_v.at[b], sg[b])

        def wait_gather(b):
            pltpu.make_async_copy(
                table_s.at[idx_v.at[b]], rows_v.at[b], sg[b]).wait()

        def fire_out(b, chunk):
            pltpu.async_copy(
                rows_v.at[b], out_hbm.at[pl.ds((irow0 + chunk) * _C, _C)],
                so[b])

        def wait_out(b, chunk):
            pltpu.make_async_copy(
                rows_v.at[b], out_hbm.at[pl.ds((irow0 + chunk) * _C, _C)],
                so[b]).wait()

        for c in range(_LEAD):
            load_ids(c, c)
            fire_gather(c)

        lag = _NBUF - _LEAD

        def body(q, carry):
            c0 = q * _NBUF
            for b in range(_NBUF):
                c = c0 + b
                wait_gather(b)
                fire_out(b, c)
                bn = (b + _LEAD) % _NBUF
                @pl.when(c >= lag)
                def _():
                    wait_out(bn, c - lag)

                @pl.when(c + _LEAD < chunks)
                def _():
                    load_ids(bn, c + _LEAD)
                    fire_gather(bn)
            return carry

        lax.fori_loop(0, quads, body, 0)
        for k in range(lag):
            c = chunks - lag + k
            wait_out(c % _NBUF, c)

    return run(ids2d, table)


def kernel(input_ids, attention_mask, emb_table):
    del attention_mask
    b, l = input_ids.shape
    n = b * l
    ids2d = input_ids.astype(jnp.int32).reshape(n // _C, _C)
    out = _sc_embed(ids2d, emb_table)
    return out.reshape(b, l, _HID)
